# Initial kernel scaffold; baseline (speedup 1.0000x reference)
import jax
import jax.numpy as jnp
from jax.experimental import pallas as pl


def kernel(x, edge, W_in, b_in, W_g1, b_g1, W_g2, b_g2, W_out, b_out):
    raise NotImplementedError("write your pallas kernel here")


# speedup vs baseline: 8.2149x; 8.2149x over previous
import functools

import jax
import jax.numpy as jnp
from jax import lax
from jax.experimental import pallas as pl
from jax.experimental.pallas import tpu as pltpu
from jax.experimental.pallas import tpu_sc as plsc

N = 10000
E = 320000
C = 128
NC = 2
NS = 16
NW = NC * NS
CHUNK = 128
CPT = 80
EPAD = NW * CPT * CHUNK
NPAD = 10240
RPT = NPAD // NS
DEGW = 128
HALF = 40
NBUF = 2


def _build_rid(rid_v, base):
    for j in range(CPT // 16):
        rid_v[pl.ds(j * 16, 16)] = base + j * 16 + lax.iota(jnp.int32, 16)


def _sc_degree_body(col_hbm, out_hbm, rid_v, idxc_v, ones_v, deg_sh, sems):
    c = lax.axis_index("c")
    s = lax.axis_index("s")
    wid = c * NS + s

    _build_rid(rid_v, wid * CPT)
    pltpu.async_copy(col_hbm.at[rid_v], idxc_v, sems).wait()

    def zfill(i, carry):
        for j in range(DEGW // 16):
            ones_v[i, pl.ds(j * 16, 16)] = jnp.zeros((16,), jnp.float32)
        return carry

    lax.fori_loop(0, CHUNK, zfill, 0)
    for t in range(RPT // CHUNK):
        pltpu.sync_copy(ones_v, deg_sh.at[pl.ds(s * RPT + t * CHUNK, CHUNK)])

    def fill(i, carry):
        for j in range(DEGW // 16):
            ones_v[i, pl.ds(j * 16, 16)] = jnp.full((16,), 1.0, jnp.float32)
        return carry

    lax.fori_loop(0, CHUNK, fill, 0)
    plsc.subcore_barrier()

    def issue(k, carry):
        @pl.when(k >= 8)
        def _():
            pltpu.make_async_copy(ones_v, deg_sh.at[idxc_v.at[k - 8]],
                                  sems).wait()

        pltpu.async_copy(ones_v, deg_sh.at[idxc_v.at[k]], sems, add=True)
        return carry

    lax.fori_loop(0, CPT, issue, 0)

    def drain(k, carry):
        pltpu.make_async_copy(ones_v, deg_sh.at[idxc_v.at[k]], sems).wait()
        return carry

    lax.fori_loop(CPT - 8, CPT, drain, 0)
    plsc.subcore_barrier()
    for t in range(RPT // CHUNK):
        pltpu.sync_copy(
            deg_sh.at[pl.ds(s * RPT + t * CHUNK, CHUNK)],
            out_hbm.at[pl.ds(c * NPAD + s * RPT + t * CHUNK, CHUNK)])


def _sc_aggregate_body(pk_hbm, z_hbm, out_hbm,
                       rid_v, pk_v, ir0, ir1, ic0, ic1, b0, b1, s_sh,
                       semg, sems, sempk):
    c = lax.axis_index("c")
    s = lax.axis_index("s")
    wid = c * NS + s
    irs = [ir0, ir1]
    ics = [ic0, ic1]
    bufs = [b0, b1]

    _build_rid(rid_v, wid * CPT)

    def zfill(i, carry):
        for j in range(C // 16):
            b0[i, pl.ds(j * 16, 16)] = jnp.zeros((16,), jnp.float32)
        return carry

    lax.fori_loop(0, CHUNK, zfill, 0)
    for t in range(RPT // CHUNK):
        pltpu.sync_copy(b0, s_sh.at[pl.ds(s * RPT + t * CHUNK, CHUNK)])
    plsc.subcore_barrier()

    def run_ring(base):
        def step(q, carry):
            for b in range(NBUF):
                k = base + q * NBUF + b

                @pl.when((k >= NBUF) & (k - NBUF < CPT))
                def _():
                    pltpu.make_async_copy(bufs[b], s_sh.at[ics[b]],
                                          sems).wait()

                def unpack(j, carry2):
                    p = pk_v[(k - base), pl.ds(j * 16, 16)]
                    irs[b][pl.ds(j * 16, 16)] = lax.shift_right_logical(
                        p, jnp.int32(14))
                    ics[b][pl.ds(j * 16, 16)] = p & jnp.int32(16383)
                    return carry2

                lax.fori_loop(0, CHUNK // 16, unpack, 0)
                pltpu.async_copy(z_hbm.at[irs[b]], bufs[b], semg)

                bp = (b + NBUF - 1) % NBUF
                kp = k - 1

                @pl.when((kp >= 0) & (kp < CPT))
                def _():
                    pltpu.make_async_copy(z_hbm.at[irs[bp]], bufs[bp],
                                          semg).wait()
                    pltpu.async_copy(bufs[bp], s_sh.at[ics[bp]], sems,
                                     add=True)

            return carry

        return step

    for h in range(CPT // HALF):
        pltpu.async_copy(
            pk_hbm.at[rid_v.at[pl.ds(h * HALF, HALF)]], pk_v, sempk).wait()
        lax.fori_loop(0, HALF // NBUF, run_ring(h * HALF), 0)

    bl = (CPT - 1) % NBUF
    pltpu.make_async_copy(z_hbm.at[irs[bl]], bufs[bl], semg).wait()
    pltpu.async_copy(bufs[bl], s_sh.at[ics[bl]], sems, add=True)
    for b in range(NBUF):
        pltpu.make_async_copy(bufs[b], s_sh.at[ics[b]], sems).wait()
    plsc.subcore_barrier()
    for t in range(RPT // CHUNK):
        pltpu.sync_copy(
            s_sh.at[pl.ds(s * RPT + t * CHUNK, CHUNK)],
            out_hbm.at[pl.ds(c * NPAD + s * RPT + t * CHUNK, CHUNK)])


@functools.cache
def _sc_kernels():
    mesh = plsc.VectorSubcoreMesh(core_axis_name="c", subcore_axis_name="s")
    rid = pltpu.VMEM((CPT,), jnp.int32)
    idx2 = pltpu.VMEM((CPT, CHUNK), jnp.int32)
    deg = pl.kernel(
        _sc_degree_body,
        out_type=jax.ShapeDtypeStruct((NC * NPAD, DEGW), jnp.float32),
        scratch_types=[
            rid, idx2,
            pltpu.VMEM((CHUNK, DEGW), jnp.float32),
            pltpu.VMEM_SHARED((NPAD, DEGW), jnp.float32),
            pltpu.SemaphoreType.DMA,
        ],
        mesh=mesh,
    )
    buf = pltpu.VMEM((CHUNK, C), jnp.float32)
    idxc = pltpu.VMEM((CHUNK,), jnp.int32)
    agg = pl.kernel(
        _sc_aggregate_body,
        out_type=jax.ShapeDtypeStruct((NC * NPAD, C), jnp.float32),
        scratch_types=[
            rid,
            pltpu.VMEM((HALF, CHUNK), jnp.int32),
            idxc, idxc, idxc, idxc,
            buf, buf,
            pltpu.VMEM_SHARED((NPAD, C), jnp.float32),
            pltpu.SemaphoreType.DMA,
            pltpu.SemaphoreType.DMA,
            pltpu.SemaphoreType.DMA,
        ],
        mesh=mesh,
    )
    return deg, agg


def _sc_degree(col2d):
    return _sc_kernels()[0](col2d)


def _sc_aggregate(pk2d, z):
    return _sc_kernels()[1](pk2d, z)


BLK = 1000


def _dinv_block(da_ref, db_ref):
    deg = da_ref[:, 0:1] + db_ref[:, 0:1] + 1.0
    return lax.rsqrt(deg)


def _tc1_body(x_ref, win_ref, bin_ref, wg1_ref, da_ref, db_ref,
              h0_ref, z1_ref):
    dinv = _dinv_block(da_ref, db_ref)
    h0 = jnp.maximum(
        jnp.dot(x_ref[...], win_ref[...], preferred_element_type=jnp.float32)
        + bin_ref[...], 0.0)
    h0_ref[...] = h0
    z1_ref[...] = dinv * jnp.dot(h0, wg1_ref[...],
                                 preferred_element_type=jnp.float32)


def _tc2_body(s1a_ref, s1b_ref, z1_ref, da_ref, db_ref, wg2_ref, bg1_ref,
              z2_ref):
    dinv = _dinv_block(da_ref, db_ref)
    h1 = jnp.maximum(
        dinv * (s1a_ref[...] + s1b_ref[...] + z1_ref[...]) + bg1_ref[...],
        0.0)
    z2_ref[...] = dinv * jnp.dot(h1, wg2_ref[...],
                                 preferred_element_type=jnp.float32)


def _tc3_body(s2a_ref, s2b_ref, z2_ref, da_ref, db_ref, h0_ref,
              wout_ref, bout_ref, bg2_ref, hf_ref, out_ref):
    dinv = _dinv_block(da_ref, db_ref)
    conv = dinv * (s2a_ref[...] + s2b_ref[...] + z2_ref[...]) + bg2_ref[...]
    hf = jnp.maximum(conv + h0_ref[...], 0.0)
    hf_ref[...] = hf
    out_ref[...] = (jnp.dot(hf, wout_ref[...],
                            preferred_element_type=jnp.float32)
                    + bout_ref[...])


def _rows(shape):
    return pl.BlockSpec(shape, lambda i: (i, 0))


def _full(shape):
    return pl.BlockSpec(shape, lambda i: (0, 0))


_MAT = jax.ShapeDtypeStruct((N, C), jnp.float32)
_GRID = (N // BLK,)


def _tc1(x, W_in, b_in2, W_g1, da, db):
    return pl.pallas_call(
        _tc1_body,
        grid=_GRID,
        in_specs=[_rows((BLK, C)), _full((C, C)), _full((1, C)),
                  _full((C, C)), _rows((BLK, DEGW)), _rows((BLK, DEGW))],
        out_specs=[_rows((BLK, C)), _rows((BLK, C))],
        out_shape=[_MAT, _MAT],
    )(x, W_in, b_in2, W_g1, da, db)


def _tc2(s1a, s1b, z1, da, db, W_g2, bg1_2):
    return pl.pallas_call(
        _tc2_body,
        grid=_GRID,
        in_specs=[_rows((BLK, C)), _rows((BLK, C)), _rows((BLK, C)),
                  _rows((BLK, DEGW)), _rows((BLK, DEGW)),
                  _full((C, C)), _full((1, C))],
        out_specs=[_rows((BLK, C))],
        out_shape=[_MAT],
    )(s1a, s1b, z1, da, db, W_g2, bg1_2)


def _tc3(s2a, s2b, z2, da, db, h0, W_out, bout_2, bg2_2):
    return pl.pallas_call(
        _tc3_body,
        grid=_GRID,
        in_specs=[_rows((BLK, C)), _rows((BLK, C)), _rows((BLK, C)),
                  _rows((BLK, DEGW)), _rows((BLK, DEGW)), _rows((BLK, C)),
                  _full((C, C)), _full((1, C)), _full((1, C))],
        out_specs=[_rows((BLK, C)), _rows((BLK, C))],
        out_shape=[_MAT, _MAT],
    )(s2a, s2b, z2, da, db, h0, W_out, bout_2, bg2_2)


def kernel(x, edge, W_in, b_in, W_g1, b_g1, W_g2, b_g2, W_out, b_out):
    row = edge[0]
    col = edge[1]
    npad_e = EPAD - E
    row2d = jnp.concatenate(
        [row, jnp.zeros((npad_e,), row.dtype)]).reshape(-1, CHUNK)
    col2d = jnp.concatenate(
        [col, jnp.full((npad_e,), NPAD - 1, col.dtype)]).reshape(-1, CHUNK)

    pk2d = (row2d * jnp.int32(16384) + col2d)

    deg2 = _sc_degree(col2d)
    da, db = deg2[:N], deg2[NPAD:NPAD + N]

    h0, z1 = _tc1(x, W_in, b_in.reshape(1, C), W_g1, da, db)

    s1 = _sc_aggregate(pk2d, z1)
    z2, = _tc2(s1[:N], s1[NPAD:NPAD + N], z1, da, db, W_g2,
               b_g1.reshape(1, C))

    s2 = _sc_aggregate(pk2d, z2)
    hf, out = _tc3(s2[:N], s2[NPAD:NPAD + N], z2, da, db, h0,
                   W_out, b_out.reshape(1, C), b_g2.reshape(1, C))
    return hf, out


# speedup vs baseline: 9.4782x; 1.1538x over previous
import functools

import jax
import jax.numpy as jnp
from jax import lax
from jax.experimental import pallas as pl
from jax.experimental.pallas import tpu as pltpu
from jax.experimental.pallas import tpu_sc as plsc

N = 10000
E = 320000
C = 128
NC = 2
NS = 16
NW = NC * NS
CHUNK = 128
CPT = 80
CPT0 = 120
CPT1 = 40
CPTR = 128
EPAD = NW * CPT * CHUNK
NPAD = 10240
RPT = NPAD // NS
DEGW = 128
HALF = 40
NBUF = 2


def _build_rid(rid_v, base):
    for j in range(CPT // 16):
        rid_v[pl.ds(j * 16, 16)] = base + j * 16 + lax.iota(jnp.int32, 16)


def _sc_degree_body(col_hbm, out_hbm, rid_v, idxc_v, ones_v, deg_sh, sems):
    c = lax.axis_index("c")
    s = lax.axis_index("s")
    wid = c * NS + s

    _build_rid(rid_v, wid * CPT)
    pltpu.async_copy(col_hbm.at[rid_v], idxc_v, sems).wait()

    def zfill(i, carry):
        for j in range(DEGW // 16):
            ones_v[i, pl.ds(j * 16, 16)] = jnp.zeros((16,), jnp.float32)
        return carry

    lax.fori_loop(0, CHUNK, zfill, 0)
    for t in range(RPT // CHUNK):
        pltpu.sync_copy(ones_v, deg_sh.at[pl.ds(s * RPT + t * CHUNK, CHUNK)])

    def fill(i, carry):
        for j in range(DEGW // 16):
            ones_v[i, pl.ds(j * 16, 16)] = jnp.full((16,), 1.0, jnp.float32)
        return carry

    lax.fori_loop(0, CHUNK, fill, 0)
    plsc.subcore_barrier()

    def issue(k, carry):
        @pl.when(k >= 8)
        def _():
            pltpu.make_async_copy(ones_v, deg_sh.at[idxc_v.at[k - 8]],
                                  sems).wait()

        pltpu.async_copy(ones_v, deg_sh.at[idxc_v.at[k]], sems, add=True)
        return carry

    lax.fori_loop(0, CPT, issue, 0)

    def drain(k, carry):
        pltpu.make_async_copy(ones_v, deg_sh.at[idxc_v.at[k]], sems).wait()
        return carry

    lax.fori_loop(CPT - 8, CPT, drain, 0)
    plsc.subcore_barrier()
    for t in range(RPT // CHUNK):
        pltpu.sync_copy(
            deg_sh.at[pl.ds(s * RPT + t * CHUNK, CHUNK)],
            out_hbm.at[pl.ds(c * NPAD + s * RPT + t * CHUNK, CHUNK)])


def _sc_aggregate_body(pk_hbm, z_hbm, out_hbm,
                       rid_v, pk_v, ir0, ir1, ic0, ic1, b0, b1, s_sh,
                       semg, sems, sempk):
    c = lax.axis_index("c")
    s = lax.axis_index("s")
    wid = c * NS + s
    irs = [ir0, ir1]
    ics = [ic0, ic1]
    bufs = [b0, b1]

    cpt_c = jnp.where(c == 0, CPT0, CPT1)
    base_c = jnp.where(c == 0, s * CPT0, NS * CPT0 + s * CPT1)
    for j in range(CPTR // 16):
        rid_v[pl.ds(j * 16, 16)] = jnp.minimum(
            base_c + j * 16 + lax.iota(jnp.int32, 16),
            jnp.int32(NS * (CPT0 + CPT1) - 1))

    def zfill(i, carry):
        for j in range(C // 16):
            b0[i, pl.ds(j * 16, 16)] = jnp.zeros((16,), jnp.float32)
        return carry

    lax.fori_loop(0, CHUNK, zfill, 0)
    for t in range(RPT // CHUNK):
        pltpu.sync_copy(b0, s_sh.at[pl.ds(s * RPT + t * CHUNK, CHUNK)])
    plsc.subcore_barrier()

    def run_ring(base):
        def step(q, carry):
            for b in range(NBUF):
                k = base + q * NBUF + b

                @pl.when((k >= NBUF) & (k - NBUF < cpt_c))
                def _():
                    pltpu.make_async_copy(bufs[b], s_sh.at[ics[b]],
                                          sems).wait()

                @pl.when(k < cpt_c)
                def _():
                    def unpack(j, carry2):
                        p = pk_v[(k - base), pl.ds(j * 16, 16)]
                        irs[b][pl.ds(j * 16, 16)] = lax.shift_right_logical(
                            p, jnp.int32(14))
                        ics[b][pl.ds(j * 16, 16)] = p & jnp.int32(16383)
                        return carry2

                    lax.fori_loop(0, CHUNK // 16, unpack, 0)
                    pltpu.async_copy(z_hbm.at[irs[b]], bufs[b], semg)

                bp = (b + NBUF - 1) % NBUF
                kp = k - 1

                @pl.when((kp >= 0) & (kp < cpt_c))
                def _():
                    pltpu.make_async_copy(z_hbm.at[irs[bp]], bufs[bp],
                                          semg).wait()
                    pltpu.async_copy(bufs[bp], s_sh.at[ics[bp]], sems,
                                     add=True)

            return carry

        return step

    for h in range(CPT0 // HALF):
        @pl.when(h * HALF < cpt_c)
        def _():
            pltpu.async_copy(
                pk_hbm.at[rid_v.at[pl.ds(h * HALF, HALF)]], pk_v,
                sempk).wait()

        lax.fori_loop(0, HALF // NBUF, run_ring(h * HALF), 0)

    lax.fori_loop(0, 1, run_ring(CPT0), 0)
    plsc.subcore_barrier()
    for t in range(RPT // CHUNK):
        pltpu.sync_copy(
            s_sh.at[pl.ds(s * RPT + t * CHUNK, CHUNK)],
            out_hbm.at[pl.ds(c * NPAD + s * RPT + t * CHUNK, CHUNK)])


@functools.cache
def _sc_kernels():
    mesh = plsc.VectorSubcoreMesh(core_axis_name="c", subcore_axis_name="s")
    rid = pltpu.VMEM((CPT,), jnp.int32)
    idx2 = pltpu.VMEM((CPT, CHUNK), jnp.int32)
    deg = pl.kernel(
        _sc_degree_body,
        out_type=jax.ShapeDtypeStruct((NC * NPAD, DEGW), jnp.float32),
        scratch_types=[
            rid, idx2,
            pltpu.VMEM((CHUNK, DEGW), jnp.float32),
            pltpu.VMEM_SHARED((NPAD, DEGW), jnp.float32),
            pltpu.SemaphoreType.DMA,
        ],
        mesh=mesh,
    )
    buf = pltpu.VMEM((CHUNK, C), jnp.float32)
    idxc = pltpu.VMEM((CHUNK,), jnp.int32)
    agg = pl.kernel(
        _sc_aggregate_body,
        out_type=jax.ShapeDtypeStruct((NC * NPAD, C), jnp.float32),
        scratch_types=[
            pltpu.VMEM((CPTR,), jnp.int32),
            pltpu.VMEM((HALF, CHUNK), jnp.int32),
            idxc, idxc, idxc, idxc,
            buf, buf,
            pltpu.VMEM_SHARED((NPAD, C), jnp.float32),
            pltpu.SemaphoreType.DMA,
            pltpu.SemaphoreType.DMA,
            pltpu.SemaphoreType.DMA,
        ],
        mesh=mesh,
    )
    return deg, agg


def _sc_degree(col2d):
    return _sc_kernels()[0](col2d)


def _sc_aggregate(pk2d, z):
    return _sc_kernels()[1](pk2d, z)


BLK = 1000


def _dinv_block(da_ref, db_ref):
    deg = da_ref[:, 0:1] + db_ref[:, 0:1] + 1.0
    return lax.rsqrt(deg)


def _tc1a_body(x_ref, win_ref, bin_ref, h0_ref):
    h0_ref[...] = jnp.maximum(
        jnp.dot(x_ref[...], win_ref[...], preferred_element_type=jnp.float32)
        + bin_ref[...], 0.0)


def _tc1b_body(h0_ref, wg1_ref, da_ref, db_ref, z1_ref):
    dinv = _dinv_block(da_ref, db_ref)
    z1_ref[...] = dinv * jnp.dot(h0_ref[...], wg1_ref[...],
                                 preferred_element_type=jnp.float32)


def _tc2_body(s1a_ref, s1b_ref, z1_ref, da_ref, db_ref, wg2_ref, bg1_ref,
              z2_ref):
    dinv = _dinv_block(da_ref, db_ref)
    h1 = jnp.maximum(
        dinv * (s1a_ref[...] + s1b_ref[...] + z1_ref[...]) + bg1_ref[...],
        0.0)
    z2_ref[...] = dinv * jnp.dot(h1, wg2_ref[...],
                                 preferred_element_type=jnp.float32)


def _tc3_body(s2a_ref, s2b_ref, z2_ref, da_ref, db_ref, h0_ref,
              wout_ref, bout_ref, bg2_ref, hf_ref, out_ref):
    dinv = _dinv_block(da_ref, db_ref)
    conv = dinv * (s2a_ref[...] + s2b_ref[...] + z2_ref[...]) + bg2_ref[...]
    hf = jnp.maximum(conv + h0_ref[...], 0.0)
    hf_ref[...] = hf
    out_ref[...] = (jnp.dot(hf, wout_ref[...],
                            preferred_element_type=jnp.float32)
                    + bout_ref[...])


def _rows(shape):
    return pl.BlockSpec(shape, lambda i: (i, 0))


def _full(shape):
    return pl.BlockSpec(shape, lambda i: (0, 0))


_MAT = jax.ShapeDtypeStruct((N, C), jnp.float32)
_GRID = (N // BLK,)


def _tc1a(x, W_in, b_in2):
    return pl.pallas_call(
        _tc1a_body,
        grid=_GRID,
        in_specs=[_rows((BLK, C)), _full((C, C)), _full((1, C))],
        out_specs=[_rows((BLK, C))],
        out_shape=[_MAT],
    )(x, W_in, b_in2)


def _tc1b(h0, W_g1, da, db):
    return pl.pallas_call(
        _tc1b_body,
        grid=_GRID,
        in_specs=[_rows((BLK, C)), _full((C, C)),
                  _rows((BLK, DEGW)), _rows((BLK, DEGW))],
        out_specs=[_rows((BLK, C))],
        out_shape=[_MAT],
    )(h0, W_g1, da, db)


def _tc2(s1a, s1b, z1, da, db, W_g2, bg1_2):
    return pl.pallas_call(
        _tc2_body,
        grid=_GRID,
        in_specs=[_rows((BLK, C)), _rows((BLK, C)), _rows((BLK, C)),
                  _rows((BLK, DEGW)), _rows((BLK, DEGW)),
                  _full((C, C)), _full((1, C))],
        out_specs=[_rows((BLK, C))],
        out_shape=[_MAT],
    )(s1a, s1b, z1, da, db, W_g2, bg1_2)


def _tc3(s2a, s2b, z2, da, db, h0, W_out, bout_2, bg2_2):
    return pl.pallas_call(
        _tc3_body,
        grid=_GRID,
        in_specs=[_rows((BLK, C)), _rows((BLK, C)), _rows((BLK, C)),
                  _rows((BLK, DEGW)), _rows((BLK, DEGW)), _rows((BLK, C)),
                  _full((C, C)), _full((1, C)), _full((1, C))],
        out_specs=[_rows((BLK, C)), _rows((BLK, C))],
        out_shape=[_MAT, _MAT],
    )(s2a, s2b, z2, da, db, h0, W_out, bout_2, bg2_2)


def kernel(x, edge, W_in, b_in, W_g1, b_g1, W_g2, b_g2, W_out, b_out):
    row = edge[0]
    col = edge[1]
    npad_e = EPAD - E
    row2d = jnp.concatenate(
        [row, jnp.zeros((npad_e,), row.dtype)]).reshape(-1, CHUNK)
    col2d = jnp.concatenate(
        [col, jnp.full((npad_e,), NPAD - 1, col.dtype)]).reshape(-1, CHUNK)

    pk2d = (row2d * jnp.int32(16384) + col2d)

    deg2 = _sc_degree(col2d)
    da, db = deg2[:N], deg2[NPAD:NPAD + N]

    h0, = _tc1a(x, W_in, b_in.reshape(1, C))
    z1, = _tc1b(h0, W_g1, da, db)

    s1 = _sc_aggregate(pk2d, z1)
    z2, = _tc2(s1[:N], s1[NPAD:NPAD + N], z1, da, db, W_g2,
               b_g1.reshape(1, C))

    s2 = _sc_aggregate(pk2d, z2)
    hf, out = _tc3(s2[:N], s2[NPAD:NPAD + N], z2, da, db, h0,
                   W_out, b_out.reshape(1, C), b_g2.reshape(1, C))
    return hf, out
